# baseline (speedup 1.0000x reference)
import functools

import jax
import jax.numpy as jnp
from jax import lax
from jax.experimental import pallas as pl
from jax.experimental.pallas import tpu as pltpu
from jax.experimental.pallas import tpu_sc as plsc

N = 10000
E = 320000
D = 128
NCHUNK = 4
NC, NS = 2, 16
NW = NC * NS
N2 = 10240
RPT = N2 // NS
EPW = 10240
E2 = NW * EPW
BATCH = 64
NSEG = 5
NBS = 32
PREP_R = 1280


def _prep_body(ns_ref, w_ref, b_ref, out_ref):
    out_ref[0] = (
        jnp.dot(ns_ref[...], w_ref[...], preferred_element_type=jnp.float32)
        + b_ref[0, 0][None, :]
    )


def _prep(ns, wbig, bias):
    return pl.pallas_call(
        _prep_body,
        grid=(8, N2 // PREP_R),
        in_specs=[
            pl.BlockSpec((PREP_R, D), lambda c, i: (i, 0)),
            pl.BlockSpec((D, 128), lambda c, i: (0, c)),
            pl.BlockSpec((1, 1, 128), lambda c, i: (c, 0, 0)),
        ],
        out_specs=pl.BlockSpec((1, PREP_R, 128), lambda c, i: (c, i, 0)),
        out_shape=jax.ShapeDtypeStruct((8, N2, 128), jnp.float32),
    )(ns, wbig, bias)


def _sc_body(a0, a1, a2, a3, b0, b1, b2, b3, fi_hbm, ti_hbm, out_hbm,
             fi_v, ti_v, ba0, ba1, bb0, bb1, m0,
             sa0, sa1, sb0, sb1, acc):
    a_tab = (a0, a1, a2, a3)
    b_tab = (b0, b1, b2, b3)
    cid = lax.axis_index("c")
    sid = lax.axis_index("s")
    wid = sid * NC + cid
    zero = jnp.zeros((16,), jnp.float32)

    def compute(ba, bb, m):
        @plsc.parallel_loop(0, BATCH, unroll=4)
        def _row(r):
            for c8 in range(8):
                s_ = pl.ds(16 * c8, 16)
                m[r, s_] = jnp.maximum(ba[r, s_] + bb[r, s_], zero)

    for p in range(NCHUNK):
        def zrow(r, carry):
            for c8 in range(8):
                m0[r, pl.ds(16 * c8, 16)] = zero
            return carry

        lax.fori_loop(0, BATCH, zrow, 0)
        for z in range(RPT // BATCH):
            pltpu.sync_copy(m0, acc.at[pl.ds(sid * RPT + z * BATCH, BATCH)])
        plsc.subcore_barrier()

        sc_ref = ti_v if p < 2 else fi_v
        at_, bt_ = a_tab[p], b_tab[p]

        def seg(s5, scarry):
            pltpu.sync_copy(fi_hbm.at[wid, s5], fi_v)
            pltpu.sync_copy(ti_hbm.at[wid, s5], ti_v)
            pltpu.async_copy(at_.at[fi_v.at[0]], ba0, sa0)
            pltpu.async_copy(bt_.at[ti_v.at[0]], bb0, sb0)
            pltpu.async_copy(at_.at[fi_v.at[1]], ba1, sa1)
            pltpu.async_copy(bt_.at[ti_v.at[1]], bb1, sb1)

            def it(j2, carry):
                j0 = 2 * j2
                pltpu.make_async_copy(at_.at[fi_v.at[j0]], ba0, sa0).wait()
                pltpu.make_async_copy(bt_.at[ti_v.at[j0]], bb0, sb0).wait()
                compute(ba0, bb0, m0)

                @pl.when(j2 < NBS // 2 - 1)
                def _prefetch0():
                    pltpu.async_copy(at_.at[fi_v.at[j0 + 2]], ba0, sa0)
                    pltpu.async_copy(bt_.at[ti_v.at[j0 + 2]], bb0, sb0)

                pltpu.sync_copy(m0, acc.at[sc_ref.at[j0]], add=True)
                pltpu.make_async_copy(at_.at[fi_v.at[j0 + 1]], ba1, sa1).wait()
                pltpu.make_async_copy(bt_.at[ti_v.at[j0 + 1]], bb1, sb1).wait()
                compute(ba1, bb1, m0)

                @pl.when(j2 < NBS // 2 - 1)
                def _prefetch1():
                    pltpu.async_copy(at_.at[fi_v.at[j0 + 3]], ba1, sa1)
                    pltpu.async_copy(bt_.at[ti_v.at[j0 + 3]], bb1, sb1)

                pltpu.sync_copy(m0, acc.at[sc_ref.at[j0 + 1]], add=True)
                return carry

            lax.fori_loop(0, NBS // 2, it, 0)
            return scarry

        lax.fori_loop(0, NSEG, seg, 0)
        plsc.subcore_barrier()
        pltpu.sync_copy(
            acc.at[pl.ds(sid * RPT, RPT)],
            out_hbm.at[cid, p, pl.ds(sid * RPT, RPT)],
        )
        plsc.subcore_barrier()


@functools.lru_cache(maxsize=1)
def _get_sc_kernel():
  return pl.kernel(
    _sc_body,
    out_type=jax.ShapeDtypeStruct((NC, NCHUNK, N2, 128), jnp.float32),
    mesh=plsc.VectorSubcoreMesh(core_axis_name="c", subcore_axis_name="s",
                                num_cores=NC, num_subcores=NS),
    scratch_types=[
        pltpu.VMEM((NBS, BATCH), jnp.int32),
        pltpu.VMEM((NBS, BATCH), jnp.int32),
        pltpu.VMEM((BATCH, 128), jnp.float32),
        pltpu.VMEM((BATCH, 128), jnp.float32),
        pltpu.VMEM((BATCH, 128), jnp.float32),
        pltpu.VMEM((BATCH, 128), jnp.float32),
        pltpu.VMEM((BATCH, 128), jnp.float32),
        pltpu.SemaphoreType.DMA,
        pltpu.SemaphoreType.DMA,
        pltpu.SemaphoreType.DMA,
        pltpu.SemaphoreType.DMA,
        pltpu.VMEM_SHARED((N2, 128), jnp.float32),
    ],
  )


def _post_body(hc_ref, ns_ref, w2p_ref, wr2p_ref, wn1a_ref, wn1b_ref,
               bn1_ref, wn2_ref, bn2_ref, out_ref):
    hf = jnp.concatenate(
        [hc_ref[0, 0] + hc_ref[1, 0], hc_ref[0, 1] + hc_ref[1, 1]], axis=-1)
    hr = jnp.concatenate(
        [hc_ref[0, 2] + hc_ref[1, 2], hc_ref[0, 3] + hc_ref[1, 3]], axis=-1)
    agg = (jnp.dot(hf, w2p_ref[...], preferred_element_type=jnp.float32)
           + jnp.dot(hr, wr2p_ref[...], preferred_element_type=jnp.float32))
    ns = ns_ref[...]
    u = jnp.maximum(
        jnp.dot(agg, wn1a_ref[...], preferred_element_type=jnp.float32)
        + jnp.dot(ns, wn1b_ref[...], preferred_element_type=jnp.float32)
        + bn1_ref[0][None, :], 0.0)
    out_ref[...] = (ns + jnp.dot(u, wn2_ref[...],
                                 preferred_element_type=jnp.float32)
                    + bn2_ref[0][None, :])


def _post(hc, ns, w2p, wr2p, wn1a, wn1b, bn1, wn2, bn2):
    R = 1000
    return pl.pallas_call(
        _post_body,
        grid=(N // R,),
        in_specs=[
            pl.BlockSpec((NC, NCHUNK, R, 128), lambda i: (0, 0, i, 0)),
            pl.BlockSpec((R, D), lambda i: (i, 0)),
            pl.BlockSpec((256, 256), lambda i: (0, 0)),
            pl.BlockSpec((256, 256), lambda i: (0, 0)),
            pl.BlockSpec((256, 256), lambda i: (0, 0)),
            pl.BlockSpec((D, 256), lambda i: (0, 0)),
            pl.BlockSpec((1, 256), lambda i: (0, 0)),
            pl.BlockSpec((256, D), lambda i: (0, 0)),
            pl.BlockSpec((1, D), lambda i: (0, 0)),
        ],
        out_specs=pl.BlockSpec((R, D), lambda i: (i, 0)),
        out_shape=jax.ShapeDtypeStruct((N, D), jnp.float32),
    )(hc, ns, w2p, wr2p, wn1a, wn1b, bn1, wn2, bn2)


def kernel(node_states, from_idx, to_idx, W1, b1, W2, b2,
           Wr1, br1, Wr2, br2, Wn1, bn1, Wn2, bn2):
    f32 = jnp.float32
    zc = jnp.zeros((D, 1), f32)
    wbig = jnp.concatenate(
        [W1[:D], zc, Wr1[D:], zc, W1[D:], zc, Wr1[:D], zc], axis=1)
    half = jnp.full((1,), 0.5, f32)
    z255 = jnp.zeros((255,), f32)
    bias = jnp.concatenate(
        [z255, half, z255, half, b1, half, br1, half]).reshape(8, 1, 128)

    ns2 = jnp.concatenate(
        [node_states, jnp.zeros((N2 - N, D), f32)], axis=0)
    tabs = _prep(ns2, wbig, bias)

    pad = jnp.full((E2 - E,), N, jnp.int32) + (
        jnp.arange(E2 - E, dtype=jnp.int32) % (N2 - N))
    fi = jnp.concatenate([from_idx, pad]).reshape(NW, NSEG, NBS, BATCH)
    ti = jnp.concatenate([to_idx, pad]).reshape(NW, NSEG, NBS, BATCH)
    hc = _get_sc_kernel()(tabs[0], tabs[1], tabs[2], tabs[3],
                          tabs[4], tabs[5], tabs[6], tabs[7], fi, ti)

    w2p = jnp.concatenate([W2, b2[None]], axis=0)
    wr2p = jnp.concatenate([Wr2, br2[None]], axis=0)
    return _post(hc, node_states, w2p, wr2p, Wn1[:2 * D], Wn1[2 * D:],
                 bn1.reshape(1, 256), Wn2, bn2.reshape(1, D))


# speedup vs baseline: 1.0077x; 1.0077x over previous
import functools

import jax
import jax.numpy as jnp
from jax import lax
from jax.experimental import pallas as pl
from jax.experimental.pallas import tpu as pltpu
from jax.experimental.pallas import tpu_sc as plsc

N = 10000
E = 320000
D = 128
NCHUNK = 4
NC, NS = 2, 16
NW = NC * NS
N2 = 10240
RPT = N2 // NS
EPW = 10240
E2 = NW * EPW
BATCH = 64
NSEG = 5
NBS = 32
PREP_R = 1280


def _prep_body(ns_ref, w_ref, b_ref, out_ref):
    out_ref[0] = (
        jnp.dot(ns_ref[...], w_ref[...], preferred_element_type=jnp.float32)
        + b_ref[0, 0][None, :]
    )


def _prep(ns, wbig, bias):
    return pl.pallas_call(
        _prep_body,
        grid=(8, N2 // PREP_R),
        in_specs=[
            pl.BlockSpec((PREP_R, D), lambda c, i: (i, 0)),
            pl.BlockSpec((D, 128), lambda c, i: (0, c)),
            pl.BlockSpec((1, 1, 128), lambda c, i: (c, 0, 0)),
        ],
        out_specs=pl.BlockSpec((1, PREP_R, 128), lambda c, i: (c, i, 0)),
        out_shape=jax.ShapeDtypeStruct((8, N2, 128), jnp.float32),
    )(ns, wbig, bias)


def _sc_body(a0, a1, a2, a3, b0, b1, b2, b3, fi_hbm, ti_hbm, out_hbm,
             fi_v, ti_v, ba0, ba1, bb0, bb1, m0,
             sa0, sa1, sb0, sb1, acc):
    a_tab = (a0, a1, a2, a3)
    b_tab = (b0, b1, b2, b3)
    cid = lax.axis_index("c")
    sid = lax.axis_index("s")
    wid = sid * NC + cid
    zero = jnp.zeros((16,), jnp.float32)

    def compute(ba, bb, m):
        @plsc.parallel_loop(0, BATCH, unroll=2)
        def _row(r):
            for c8 in range(8):
                s_ = pl.ds(16 * c8, 16)
                m[r, s_] = jnp.maximum(ba[r, s_] + bb[r, s_], zero)

    for p in range(NCHUNK):
        def zrow(r, carry):
            for c8 in range(8):
                m0[r, pl.ds(16 * c8, 16)] = zero
            return carry

        lax.fori_loop(0, BATCH, zrow, 0)
        for z in range(RPT // BATCH):
            pltpu.sync_copy(m0, acc.at[pl.ds(sid * RPT + z * BATCH, BATCH)])
        plsc.subcore_barrier()

        sc_ref = ti_v if p < 2 else fi_v
        at_, bt_ = a_tab[p], b_tab[p]

        def seg(s5, scarry):
            pltpu.sync_copy(fi_hbm.at[wid, s5], fi_v)
            pltpu.sync_copy(ti_hbm.at[wid, s5], ti_v)
            pltpu.async_copy(at_.at[fi_v.at[0]], ba0, sa0)
            pltpu.async_copy(bt_.at[ti_v.at[0]], bb0, sb0)
            pltpu.async_copy(at_.at[fi_v.at[1]], ba1, sa1)
            pltpu.async_copy(bt_.at[ti_v.at[1]], bb1, sb1)

            def it(j2, carry):
                j0 = 2 * j2
                pltpu.make_async_copy(at_.at[fi_v.at[j0]], ba0, sa0).wait()
                pltpu.make_async_copy(bt_.at[ti_v.at[j0]], bb0, sb0).wait()
                compute(ba0, bb0, m0)

                @pl.when(j2 < NBS // 2 - 1)
                def _prefetch0():
                    pltpu.async_copy(at_.at[fi_v.at[j0 + 2]], ba0, sa0)
                    pltpu.async_copy(bt_.at[ti_v.at[j0 + 2]], bb0, sb0)

                pltpu.sync_copy(m0, acc.at[sc_ref.at[j0]], add=True)
                pltpu.make_async_copy(at_.at[fi_v.at[j0 + 1]], ba1, sa1).wait()
                pltpu.make_async_copy(bt_.at[ti_v.at[j0 + 1]], bb1, sb1).wait()
                compute(ba1, bb1, m0)

                @pl.when(j2 < NBS // 2 - 1)
                def _prefetch1():
                    pltpu.async_copy(at_.at[fi_v.at[j0 + 3]], ba1, sa1)
                    pltpu.async_copy(bt_.at[ti_v.at[j0 + 3]], bb1, sb1)

                pltpu.sync_copy(m0, acc.at[sc_ref.at[j0 + 1]], add=True)
                return carry

            lax.fori_loop(0, NBS // 2, it, 0)
            return scarry

        lax.fori_loop(0, NSEG, seg, 0)
        plsc.subcore_barrier()
        pltpu.sync_copy(
            acc.at[pl.ds(sid * RPT, RPT)],
            out_hbm.at[cid, p, pl.ds(sid * RPT, RPT)],
        )
        plsc.subcore_barrier()


@functools.lru_cache(maxsize=1)
def _get_sc_kernel():
  return pl.kernel(
    _sc_body,
    out_type=jax.ShapeDtypeStruct((NC, NCHUNK, N2, 128), jnp.float32),
    mesh=plsc.VectorSubcoreMesh(core_axis_name="c", subcore_axis_name="s",
                                num_cores=NC, num_subcores=NS),
    scratch_types=[
        pltpu.VMEM((NBS, BATCH), jnp.int32),
        pltpu.VMEM((NBS, BATCH), jnp.int32),
        pltpu.VMEM((BATCH, 128), jnp.float32),
        pltpu.VMEM((BATCH, 128), jnp.float32),
        pltpu.VMEM((BATCH, 128), jnp.float32),
        pltpu.VMEM((BATCH, 128), jnp.float32),
        pltpu.VMEM((BATCH, 128), jnp.float32),
        pltpu.SemaphoreType.DMA,
        pltpu.SemaphoreType.DMA,
        pltpu.SemaphoreType.DMA,
        pltpu.SemaphoreType.DMA,
        pltpu.VMEM_SHARED((N2, 128), jnp.float32),
    ],
  )


def _post_body(hc_ref, ns_ref, w2p_ref, wr2p_ref, wn1a_ref, wn1b_ref,
               bn1_ref, wn2_ref, bn2_ref, out_ref):
    hf = jnp.concatenate(
        [hc_ref[0, 0] + hc_ref[1, 0], hc_ref[0, 1] + hc_ref[1, 1]], axis=-1)
    hr = jnp.concatenate(
        [hc_ref[0, 2] + hc_ref[1, 2], hc_ref[0, 3] + hc_ref[1, 3]], axis=-1)
    agg = (jnp.dot(hf, w2p_ref[...], preferred_element_type=jnp.float32)
           + jnp.dot(hr, wr2p_ref[...], preferred_element_type=jnp.float32))
    ns = ns_ref[...]
    u = jnp.maximum(
        jnp.dot(agg, wn1a_ref[...], preferred_element_type=jnp.float32)
        + jnp.dot(ns, wn1b_ref[...], preferred_element_type=jnp.float32)
        + bn1_ref[0][None, :], 0.0)
    out_ref[...] = (ns + jnp.dot(u, wn2_ref[...],
                                 preferred_element_type=jnp.float32)
                    + bn2_ref[0][None, :])


def _post(hc, ns, w2p, wr2p, wn1a, wn1b, bn1, wn2, bn2):
    R = 1000
    return pl.pallas_call(
        _post_body,
        grid=(N // R,),
        in_specs=[
            pl.BlockSpec((NC, NCHUNK, R, 128), lambda i: (0, 0, i, 0)),
            pl.BlockSpec((R, D), lambda i: (i, 0)),
            pl.BlockSpec((256, 256), lambda i: (0, 0)),
            pl.BlockSpec((256, 256), lambda i: (0, 0)),
            pl.BlockSpec((256, 256), lambda i: (0, 0)),
            pl.BlockSpec((D, 256), lambda i: (0, 0)),
            pl.BlockSpec((1, 256), lambda i: (0, 0)),
            pl.BlockSpec((256, D), lambda i: (0, 0)),
            pl.BlockSpec((1, D), lambda i: (0, 0)),
        ],
        out_specs=pl.BlockSpec((R, D), lambda i: (i, 0)),
        out_shape=jax.ShapeDtypeStruct((N, D), jnp.float32),
    )(hc, ns, w2p, wr2p, wn1a, wn1b, bn1, wn2, bn2)


def kernel(node_states, from_idx, to_idx, W1, b1, W2, b2,
           Wr1, br1, Wr2, br2, Wn1, bn1, Wn2, bn2):
    f32 = jnp.float32
    zc = jnp.zeros((D, 1), f32)
    wbig = jnp.concatenate(
        [W1[:D], zc, Wr1[D:], zc, W1[D:], zc, Wr1[:D], zc], axis=1)
    half = jnp.full((1,), 0.5, f32)
    z255 = jnp.zeros((255,), f32)
    bias = jnp.concatenate(
        [z255, half, z255, half, b1, half, br1, half]).reshape(8, 1, 128)

    ns2 = jnp.concatenate(
        [node_states, jnp.zeros((N2 - N, D), f32)], axis=0)
    tabs = _prep(ns2, wbig, bias)

    pad = jnp.full((E2 - E,), N, jnp.int32) + (
        jnp.arange(E2 - E, dtype=jnp.int32) % (N2 - N))
    fi = jnp.concatenate([from_idx, pad]).reshape(NW, NSEG, NBS, BATCH)
    ti = jnp.concatenate([to_idx, pad]).reshape(NW, NSEG, NBS, BATCH)
    hc = _get_sc_kernel()(tabs[0], tabs[1], tabs[2], tabs[3],
                          tabs[4], tabs[5], tabs[6], tabs[7], fi, ti)

    w2p = jnp.concatenate([W2, b2[None]], axis=0)
    wr2p = jnp.concatenate([Wr2, br2[None]], axis=0)
    return _post(hc, node_states, w2p, wr2p, Wn1[:2 * D], Wn1[2 * D:],
                 bn1.reshape(1, 256), Wn2, bn2.reshape(1, D))


# speedup vs baseline: 1.0104x; 1.0027x over previous
import functools

import jax
import jax.numpy as jnp
from jax import lax
from jax.experimental import pallas as pl
from jax.experimental.pallas import tpu as pltpu
from jax.experimental.pallas import tpu_sc as plsc

N = 10000
E = 320000
D = 128
NCHUNK = 4
NC, NS = 2, 16
NW = NC * NS
N2 = 10240
RPT = N2 // NS
EPW = 10240
E2 = NW * EPW
BATCH = 64
NSEG = 5
NBS = 32
PREP_R = 1280


def _prep_body(ns_ref, w_ref, b_ref, out_ref):
    out_ref[0] = (
        jnp.dot(ns_ref[...], w_ref[...], preferred_element_type=jnp.float32)
        + b_ref[0, 0][None, :]
    )


def _prep(ns, wbig, bias):
    return pl.pallas_call(
        _prep_body,
        grid=(8, N2 // PREP_R),
        in_specs=[
            pl.BlockSpec((PREP_R, D), lambda c, i: (i, 0)),
            pl.BlockSpec((D, 128), lambda c, i: (0, c)),
            pl.BlockSpec((1, 1, 128), lambda c, i: (c, 0, 0)),
        ],
        out_specs=pl.BlockSpec((1, PREP_R, 128), lambda c, i: (c, i, 0)),
        out_shape=jax.ShapeDtypeStruct((8, N2, 128), jnp.float32),
    )(ns, wbig, bias)


def _sc_body(a0, a1, a2, a3, b0, b1, b2, b3, fi_hbm, ti_hbm, out_hbm,
             fi_v, ti_v, ba0, ba1, bb0, bb1, m0,
             sa0, sa1, sb0, sb1, acc):
    a_tab = (a0, a1, a2, a3)
    b_tab = (b0, b1, b2, b3)
    cid = lax.axis_index("c")
    sid = lax.axis_index("s")
    wid = sid * NC + cid
    zero = jnp.zeros((16,), jnp.float32)

    def compute(ba, bb, m):
        @plsc.parallel_loop(0, BATCH, unroll=2)
        def _row(r):
            for c8 in range(8):
                s_ = pl.ds(16 * c8, 16)
                m[r, s_] = jnp.maximum(ba[r, s_] + bb[r, s_], zero)

    for p in range(NCHUNK):
        def zrow(r, carry):
            for c8 in range(8):
                m0[r, pl.ds(16 * c8, 16)] = zero
            return carry

        lax.fori_loop(0, BATCH, zrow, 0)
        for z in range(RPT // BATCH):
            pltpu.async_copy(
                m0, acc.at[pl.ds(sid * RPT + z * BATCH, BATCH)], sa0)
        for z in range(RPT // BATCH):
            pltpu.make_async_copy(
                m0, acc.at[pl.ds(sid * RPT + z * BATCH, BATCH)], sa0).wait()
        plsc.subcore_barrier()

        sc_ref = ti_v if p < 2 else fi_v
        at_, bt_ = a_tab[p], b_tab[p]

        def seg(s5, scarry):
            pltpu.sync_copy(fi_hbm.at[wid, s5], fi_v)
            pltpu.sync_copy(ti_hbm.at[wid, s5], ti_v)
            pltpu.async_copy(at_.at[fi_v.at[0]], ba0, sa0)
            pltpu.async_copy(bt_.at[ti_v.at[0]], bb0, sb0)
            pltpu.async_copy(at_.at[fi_v.at[1]], ba1, sa1)
            pltpu.async_copy(bt_.at[ti_v.at[1]], bb1, sb1)

            def it(j2, carry):
                j0 = 2 * j2
                pltpu.make_async_copy(at_.at[fi_v.at[j0]], ba0, sa0).wait()
                pltpu.make_async_copy(bt_.at[ti_v.at[j0]], bb0, sb0).wait()
                compute(ba0, bb0, m0)

                @pl.when(j2 < NBS // 2 - 1)
                def _prefetch0():
                    pltpu.async_copy(at_.at[fi_v.at[j0 + 2]], ba0, sa0)
                    pltpu.async_copy(bt_.at[ti_v.at[j0 + 2]], bb0, sb0)

                pltpu.sync_copy(m0, acc.at[sc_ref.at[j0]], add=True)
                pltpu.make_async_copy(at_.at[fi_v.at[j0 + 1]], ba1, sa1).wait()
                pltpu.make_async_copy(bt_.at[ti_v.at[j0 + 1]], bb1, sb1).wait()
                compute(ba1, bb1, m0)

                @pl.when(j2 < NBS // 2 - 1)
                def _prefetch1():
                    pltpu.async_copy(at_.at[fi_v.at[j0 + 3]], ba1, sa1)
                    pltpu.async_copy(bt_.at[ti_v.at[j0 + 3]], bb1, sb1)

                pltpu.sync_copy(m0, acc.at[sc_ref.at[j0 + 1]], add=True)
                return carry

            lax.fori_loop(0, NBS // 2, it, 0)
            return scarry

        lax.fori_loop(0, NSEG, seg, 0)
        plsc.subcore_barrier()
        pltpu.sync_copy(
            acc.at[pl.ds(sid * RPT, RPT)],
            out_hbm.at[cid, p, pl.ds(sid * RPT, RPT)],
        )
        plsc.subcore_barrier()


@functools.lru_cache(maxsize=1)
def _get_sc_kernel():
  return pl.kernel(
    _sc_body,
    out_type=jax.ShapeDtypeStruct((NC, NCHUNK, N2, 128), jnp.float32),
    mesh=plsc.VectorSubcoreMesh(core_axis_name="c", subcore_axis_name="s",
                                num_cores=NC, num_subcores=NS),
    scratch_types=[
        pltpu.VMEM((NBS, BATCH), jnp.int32),
        pltpu.VMEM((NBS, BATCH), jnp.int32),
        pltpu.VMEM((BATCH, 128), jnp.float32),
        pltpu.VMEM((BATCH, 128), jnp.float32),
        pltpu.VMEM((BATCH, 128), jnp.float32),
        pltpu.VMEM((BATCH, 128), jnp.float32),
        pltpu.VMEM((BATCH, 128), jnp.float32),
        pltpu.SemaphoreType.DMA,
        pltpu.SemaphoreType.DMA,
        pltpu.SemaphoreType.DMA,
        pltpu.SemaphoreType.DMA,
        pltpu.VMEM_SHARED((N2, 128), jnp.float32),
    ],
  )


def _post_body(hc_ref, ns_ref, w2p_ref, wr2p_ref, wn1a_ref, wn1b_ref,
               bn1_ref, wn2_ref, bn2_ref, out_ref):
    hf = jnp.concatenate(
        [hc_ref[0, 0] + hc_ref[1, 0], hc_ref[0, 1] + hc_ref[1, 1]], axis=-1)
    hr = jnp.concatenate(
        [hc_ref[0, 2] + hc_ref[1, 2], hc_ref[0, 3] + hc_ref[1, 3]], axis=-1)
    agg = (jnp.dot(hf, w2p_ref[...], preferred_element_type=jnp.float32)
           + jnp.dot(hr, wr2p_ref[...], preferred_element_type=jnp.float32))
    ns = ns_ref[...]
    u = jnp.maximum(
        jnp.dot(agg, wn1a_ref[...], preferred_element_type=jnp.float32)
        + jnp.dot(ns, wn1b_ref[...], preferred_element_type=jnp.float32)
        + bn1_ref[0][None, :], 0.0)
    out_ref[...] = (ns + jnp.dot(u, wn2_ref[...],
                                 preferred_element_type=jnp.float32)
                    + bn2_ref[0][None, :])


def _post(hc, ns, w2p, wr2p, wn1a, wn1b, bn1, wn2, bn2):
    R = 1000
    return pl.pallas_call(
        _post_body,
        grid=(N // R,),
        in_specs=[
            pl.BlockSpec((NC, NCHUNK, R, 128), lambda i: (0, 0, i, 0)),
            pl.BlockSpec((R, D), lambda i: (i, 0)),
            pl.BlockSpec((256, 256), lambda i: (0, 0)),
            pl.BlockSpec((256, 256), lambda i: (0, 0)),
            pl.BlockSpec((256, 256), lambda i: (0, 0)),
            pl.BlockSpec((D, 256), lambda i: (0, 0)),
            pl.BlockSpec((1, 256), lambda i: (0, 0)),
            pl.BlockSpec((256, D), lambda i: (0, 0)),
            pl.BlockSpec((1, D), lambda i: (0, 0)),
        ],
        out_specs=pl.BlockSpec((R, D), lambda i: (i, 0)),
        out_shape=jax.ShapeDtypeStruct((N, D), jnp.float32),
    )(hc, ns, w2p, wr2p, wn1a, wn1b, bn1, wn2, bn2)


def kernel(node_states, from_idx, to_idx, W1, b1, W2, b2,
           Wr1, br1, Wr2, br2, Wn1, bn1, Wn2, bn2):
    f32 = jnp.float32
    zc = jnp.zeros((D, 1), f32)
    wbig = jnp.concatenate(
        [W1[:D], zc, Wr1[D:], zc, W1[D:], zc, Wr1[:D], zc], axis=1)
    half = jnp.full((1,), 0.5, f32)
    z255 = jnp.zeros((255,), f32)
    bias = jnp.concatenate(
        [z255, half, z255, half, b1, half, br1, half]).reshape(8, 1, 128)

    ns2 = jnp.concatenate(
        [node_states, jnp.zeros((N2 - N, D), f32)], axis=0)
    tabs = _prep(ns2, wbig, bias)

    pad = jnp.full((E2 - E,), N, jnp.int32) + (
        jnp.arange(E2 - E, dtype=jnp.int32) % (N2 - N))
    fi = jnp.concatenate([from_idx, pad]).reshape(NW, NSEG, NBS, BATCH)
    ti = jnp.concatenate([to_idx, pad]).reshape(NW, NSEG, NBS, BATCH)
    hc = _get_sc_kernel()(tabs[0], tabs[1], tabs[2], tabs[3],
                          tabs[4], tabs[5], tabs[6], tabs[7], fi, ti)

    w2p = jnp.concatenate([W2, b2[None]], axis=0)
    wr2p = jnp.concatenate([Wr2, br2[None]], axis=0)
    return _post(hc, node_states, w2p, wr2p, Wn1[:2 * D], Wn1[2 * D:],
                 bn1.reshape(1, 256), Wn2, bn2.reshape(1, D))
